# Initial kernel scaffold; baseline (speedup 1.0000x reference)
import jax
import jax.numpy as jnp
from jax.experimental import pallas as pl


def kernel(x):
    raise NotImplementedError("write your pallas kernel here")


# speedup vs baseline: 1.2517x; 1.2517x over previous
import jax
import jax.numpy as jnp
from jax.experimental import pallas as pl


_COLS = 1024


def _argmin_body(x_ref, o_ref):
    v = x_ref[0]
    mn = jnp.min(v, axis=0, keepdims=True)
    rows = jax.lax.broadcasted_iota(jnp.int32, v.shape, 0)
    big = jnp.int32(2**30)
    idx = jnp.min(jnp.where(v == mn, rows, big), axis=0)
    o_ref[0, 0] = idx


def kernel(x):
    b, k, n = x.shape
    grid = (b, n // _COLS)
    out = pl.pallas_call(
        _argmin_body,
        grid=grid,
        in_specs=[pl.BlockSpec((1, k, _COLS), lambda i, j: (i, 0, j))],
        out_specs=pl.BlockSpec((1, 1, _COLS), lambda i, j: (i, 0, j)),
        out_shape=jax.ShapeDtypeStruct((b, 1, n), jnp.int32),
    )(x)
    return out.reshape(b, n).astype(jnp.int64)
